# Initial kernel scaffold; baseline (speedup 1.0000x reference)
import jax
import jax.numpy as jnp
from jax.experimental import pallas as pl


def kernel(x, edge_index, W1_init, W1_root, b1, W2_init, W2_root, b2):
    raise NotImplementedError("write your pallas kernel here")


# speedup vs baseline: 15.3064x; 15.3064x over previous
import functools

import jax
import jax.numpy as jnp
from jax import lax
from jax.experimental import pallas as pl
from jax.experimental.pallas import tpu as pltpu
from jax.experimental.pallas import tpu_sc as plsc

N = 10000
E = 320000
D_IN = 128
D_HID = 128
D_OUT = 64

NC = 2
NS = 16
NW = NC * NS
EPW = E // NW
CHUNK = 128
NCH = E // CHUNK
RPT = 624
TAIL0 = NS * RPT
TAILN = N - TAIL0

_MESH = dict(core_axis_name="c", subcore_axis_name="s")


def _deg_body(col_hbm, out_hbm, idxbuf, acc):
    c = lax.axis_index("c")
    s = lax.axis_index("s")
    w = s * NC + c

    def zero(i, carry):
        acc[pl.ds(i * 16, 16)] = jnp.zeros((16,), jnp.float32)
        return carry

    lax.fori_loop(0, N // 16, zero, 0)

    pltpu.sync_copy(col_hbm.at[pl.ds(w * EPW, EPW)], idxbuf)
    ones = jnp.ones((16,), jnp.float32)

    def body(i, carry):
        idx = idxbuf[pl.ds(i * 16, 16)]
        plsc.addupdate_scatter(acc, [idx], ones)
        return carry

    lax.fori_loop(0, EPW // 16, body, 0)
    pltpu.sync_copy(acc, out_hbm.at[w])


_SC_PARAMS = pltpu.CompilerParams(needs_layout_passes=False)
_SC_PARAMS_LINEAR = pltpu.CompilerParams(
    needs_layout_passes=False, use_tc_tiling_on_sc=False)

_deg_call = pl.kernel(
    _deg_body,
    out_type=jax.ShapeDtypeStruct((NW, N), jnp.float32),
    mesh=plsc.VectorSubcoreMesh(**_MESH),
    compiler_params=_SC_PARAMS,
    scratch_types=[
        pltpu.VMEM((EPW,), jnp.int32),
        pltpu.VMEM((N,), jnp.float32),
    ],
)


def _agg_body(table_hbm, rowc_hbm, colc_hbm, zeros_hbm, out_hbm,
              accum, idxr, idxc, rows, sem):
    c = lax.axis_index("c")
    s = lax.axis_index("s")

    off = pl.multiple_of(s * RPT, 8)
    pltpu.sync_copy(zeros_hbm.at[pl.ds(off, RPT)], accum.at[pl.ds(off, RPT)])

    @pl.when(s == NS - 1)
    def _():
        pltpu.sync_copy(zeros_hbm.at[pl.ds(TAIL0, TAILN)],
                        accum.at[pl.ds(TAIL0, TAILN)])

    plsc.subcore_barrier()

    half = NCH // 2
    base = c * half
    nt = (half - s + NS - 1) // NS

    def body(t, carry):
        j = base + s + t * NS
        pltpu.sync_copy(rowc_hbm.at[j], idxr.at[0])
        pltpu.sync_copy(colc_hbm.at[j], idxc.at[0])
        pltpu.async_copy(table_hbm.at[idxr.at[0]], rows.at[0], sem).wait()
        pltpu.sync_copy(rows.at[0], accum.at[idxc.at[0]], add=True)
        return carry

    lax.fori_loop(0, nt, body, 0)
    plsc.subcore_barrier()
    pltpu.sync_copy(accum.at[pl.ds(off, RPT)], out_hbm.at[c, pl.ds(off, RPT)])

    @pl.when(s == NS - 1)
    def _():
        pltpu.sync_copy(accum.at[pl.ds(TAIL0, TAILN)],
                        out_hbm.at[c, pl.ds(TAIL0, TAILN)])


def _make_agg(d):
    return pl.kernel(
        _agg_body,
        out_type=jax.ShapeDtypeStruct((NC, N, d), jnp.float32),
        mesh=plsc.VectorSubcoreMesh(**_MESH),
        compiler_params=_SC_PARAMS if d % 128 == 0 else _SC_PARAMS_LINEAR,
        scratch_types=[
            pltpu.VMEM_SHARED((N, d), jnp.float32),
            pltpu.VMEM((1, CHUNK), jnp.int32),
            pltpu.VMEM((1, CHUNK), jnp.int32),
            pltpu.VMEM((1, CHUNK, d), jnp.float32),
            pltpu.SemaphoreType.DMA,
        ],
    )


_agg_hid = _make_agg(D_HID)
_agg_out = _make_agg(D_OUT)


_BN = 1024


def _tc1_body(x_ref, w1i_ref, w1r_ref, b1_ref, degp_ref,
              h0s_ref, root1_ref, dinv_ref):
    x = x_ref[...]
    deg = jnp.sum(degp_ref[...], axis=0)
    dinv = jnp.where(deg > 0, lax.rsqrt(jnp.maximum(deg, 1.0)), 0.0)
    h0 = jnp.dot(x, w1i_ref[...], preferred_element_type=jnp.float32)
    h0s_ref[...] = h0 * dinv[:, None]
    root1_ref[...] = (
        jnp.dot(x, w1r_ref[...], preferred_element_type=jnp.float32)
        + b1_ref[...][None, :]
    )
    dinv_ref[...] = dinv


def _tc1_call(x, w1i, w1r, b1, deg_parts):
    return pl.pallas_call(
        _tc1_body,
        grid=(pl.cdiv(N, _BN),),
        in_specs=[
            pl.BlockSpec((_BN, D_IN), lambda i: (i, 0)),
            pl.BlockSpec((D_IN, D_HID), lambda i: (0, 0)),
            pl.BlockSpec((D_IN, D_HID), lambda i: (0, 0)),
            pl.BlockSpec((D_HID,), lambda i: (0,)),
            pl.BlockSpec((NW, _BN), lambda i: (0, i)),
        ],
        out_specs=[
            pl.BlockSpec((_BN, D_HID), lambda i: (i, 0)),
            pl.BlockSpec((_BN, D_HID), lambda i: (i, 0)),
            pl.BlockSpec((_BN,), lambda i: (i,)),
        ],
        out_shape=[
            jax.ShapeDtypeStruct((N, D_HID), jnp.float32),
            jax.ShapeDtypeStruct((N, D_HID), jnp.float32),
            jax.ShapeDtypeStruct((N,), jnp.float32),
        ],
    )(x, w1i, w1r, b1, deg_parts)


def _tc2_body(agg_ref, root1_ref, dinv_ref, w2i_ref, w2r_ref, b2_ref,
              h1s_ref, root2_ref):
    agg = agg_ref[0] + agg_ref[1]
    dinv = dinv_ref[...]
    out1 = jnp.maximum(agg * dinv[:, None] + root1_ref[...], 0.0)
    h1 = jnp.dot(out1, w2i_ref[...], preferred_element_type=jnp.float32)
    h1s_ref[...] = h1 * dinv[:, None]
    root2_ref[...] = (
        jnp.dot(out1, w2r_ref[...], preferred_element_type=jnp.float32)
        + b2_ref[...][None, :]
    )


def _tc2_call(agg, root1, dinv, w2i, w2r, b2):
    return pl.pallas_call(
        _tc2_body,
        grid=(pl.cdiv(N, _BN),),
        in_specs=[
            pl.BlockSpec((NC, _BN, D_HID), lambda i: (0, i, 0)),
            pl.BlockSpec((_BN, D_HID), lambda i: (i, 0)),
            pl.BlockSpec((_BN,), lambda i: (i,)),
            pl.BlockSpec((D_HID, D_OUT), lambda i: (0, 0)),
            pl.BlockSpec((D_HID, D_OUT), lambda i: (0, 0)),
            pl.BlockSpec((D_OUT,), lambda i: (0,)),
        ],
        out_specs=[
            pl.BlockSpec((_BN, D_OUT), lambda i: (i, 0)),
            pl.BlockSpec((_BN, D_OUT), lambda i: (i, 0)),
        ],
        out_shape=[
            jax.ShapeDtypeStruct((N, D_OUT), jnp.float32),
            jax.ShapeDtypeStruct((N, D_OUT), jnp.float32),
        ],
    )(agg, root1, dinv, w2i, w2r, b2)


def _tc3_body(agg_ref, root2_ref, dinv_ref, out_ref):
    agg = agg_ref[0] + agg_ref[1]
    out_ref[...] = jnp.maximum(
        agg * dinv_ref[...][:, None] + root2_ref[...], 0.0)


def _tc3_call(agg, root2, dinv):
    return pl.pallas_call(
        _tc3_body,
        grid=(pl.cdiv(N, _BN),),
        in_specs=[
            pl.BlockSpec((NC, _BN, D_OUT), lambda i: (0, i, 0)),
            pl.BlockSpec((_BN, D_OUT), lambda i: (i, 0)),
            pl.BlockSpec((_BN,), lambda i: (i,)),
        ],
        out_specs=pl.BlockSpec((_BN, D_OUT), lambda i: (i, 0)),
        out_shape=jax.ShapeDtypeStruct((N, D_OUT), jnp.float32),
    )(agg, root2, dinv)


def kernel(x, edge_index, W1_init, W1_root, b1, W2_init, W2_root, b2):
    row = edge_index[0]
    col = edge_index[1]
    rowc = row.reshape(NCH, CHUNK)
    colc = col.reshape(NCH, CHUNK)
    zeros_hid = jnp.zeros((N, D_HID), jnp.float32)
    zeros_out = jnp.zeros((N, D_OUT), jnp.float32)

    deg_parts = _deg_call(col)
    h0s, root1, dinv = _tc1_call(x, W1_init, W1_root, b1, deg_parts)
    agg1 = _agg_hid(h0s, rowc, colc, zeros_hid)
    h1s, root2 = _tc2_call(agg1, root1, dinv, W2_init, W2_root, b2)
    agg2 = _agg_out(h1s, rowc, colc, zeros_out)
    return _tc3_call(agg2, root2, dinv)


# speedup vs baseline: 30.5376x; 1.9951x over previous
import functools

import jax
import jax.numpy as jnp
from jax import lax
from jax.experimental import pallas as pl
from jax.experimental.pallas import tpu as pltpu
from jax.experimental.pallas import tpu_sc as plsc

N = 10000
E = 320000
D_IN = 128
D_HID = 128
D_OUT = 64

NC = 2
NS = 16
NW = NC * NS
EPW = E // NW
CHUNK = 128
NCH = E // CHUNK
RPT = 624
TAIL0 = NS * RPT
TAILN = N - TAIL0

_MESH = dict(core_axis_name="c", subcore_axis_name="s")


def _deg_body(col_hbm, out_hbm, idxbuf, acc):
    c = lax.axis_index("c")
    s = lax.axis_index("s")
    w = s * NC + c

    def zero(i, carry):
        acc[pl.ds(i * 16, 16)] = jnp.zeros((16,), jnp.float32)
        return carry

    lax.fori_loop(0, N // 16, zero, 0)

    pltpu.sync_copy(col_hbm.at[pl.ds(w * EPW, EPW)], idxbuf)
    ones = jnp.ones((16,), jnp.float32)

    def body(i, carry):
        idx = idxbuf[pl.ds(i * 16, 16)]
        plsc.addupdate_scatter(acc, [idx], ones)
        return carry

    lax.fori_loop(0, EPW // 16, body, 0)
    pltpu.sync_copy(acc, out_hbm.at[w])


_SC_PARAMS = pltpu.CompilerParams(needs_layout_passes=False)
_SC_PARAMS_LINEAR = pltpu.CompilerParams(
    needs_layout_passes=False, use_tc_tiling_on_sc=False)

_deg_call = pl.kernel(
    _deg_body,
    out_type=jax.ShapeDtypeStruct((NW, N), jnp.float32),
    mesh=plsc.VectorSubcoreMesh(**_MESH),
    compiler_params=_SC_PARAMS,
    scratch_types=[
        pltpu.VMEM((EPW,), jnp.int32),
        pltpu.VMEM((N,), jnp.float32),
    ],
)


TPW = NCH // NS
TREM = NCH % NS
MAXT = TPW + 1
NBUF = 4


def _agg_body(table_hbm, rowc_hbm, colc_hbm, zeros_hbm, out_hbm,
              accum, idxr, idxc, rows, sems):
    c = lax.axis_index("c")
    s = lax.axis_index("s")

    off = pl.multiple_of(s * RPT, 8)
    pltpu.sync_copy(zeros_hbm.at[pl.ds(off, RPT)], accum.at[pl.ds(off, RPT)])

    @pl.when(s == NS - 1)
    def _():
        pltpu.sync_copy(zeros_hbm.at[pl.ds(TAIL0, TAILN)],
                        accum.at[pl.ds(TAIL0, TAILN)])

    extra = jnp.minimum(s, TREM)
    gstart = s * TPW + extra
    nt = jnp.where(s < TREM, MAXT, TPW)

    @pl.when(s < TREM)
    def _():
        pltpu.sync_copy(rowc_hbm.at[pl.ds(gstart, MAXT)], idxr)
        pltpu.sync_copy(colc_hbm.at[pl.ds(gstart, MAXT)], idxc)

    @pl.when(s >= TREM)
    def _():
        pltpu.sync_copy(rowc_hbm.at[pl.ds(gstart, TPW)],
                        idxr.at[pl.ds(0, TPW)])
        pltpu.sync_copy(colc_hbm.at[pl.ds(gstart, TPW)],
                        idxc.at[pl.ds(0, TPW)])

    plsc.subcore_barrier()
    table_c = table_hbm.at[c]

    for b in range(NBUF):
        pltpu.async_copy(table_c.at[idxr.at[b]], rows.at[b], sems.at[b])

    def outer(g, carry):
        for b in range(NBUF):
            t = g * NBUF + b

            @pl.when(t < nt)
            def _():
                pltpu.make_async_copy(
                    table_c.at[idxr.at[b]], rows.at[b], sems.at[b]).wait()
                pltpu.sync_copy(rows.at[b], accum.at[idxc.at[t]], add=True)

                @pl.when(t + NBUF < nt)
                def _():
                    pltpu.async_copy(table_c.at[idxr.at[t + NBUF]],
                                     rows.at[b], sems.at[b])

        return carry

    lax.fori_loop(0, (MAXT + NBUF - 1) // NBUF, outer, 0)
    plsc.subcore_barrier()
    pltpu.sync_copy(accum.at[pl.ds(off, RPT)], out_hbm.at[c, pl.ds(off, RPT)])

    @pl.when(s == NS - 1)
    def _():
        pltpu.sync_copy(accum.at[pl.ds(TAIL0, TAILN)],
                        out_hbm.at[c, pl.ds(TAIL0, TAILN)])


def _make_agg(d):
    return pl.kernel(
        _agg_body,
        out_type=jax.ShapeDtypeStruct((NC, N, d), jnp.float32),
        mesh=plsc.VectorSubcoreMesh(**_MESH),
        compiler_params=_SC_PARAMS_LINEAR,
        scratch_types=[
            pltpu.VMEM_SHARED((N, d), jnp.float32),
            pltpu.VMEM((MAXT, CHUNK), jnp.int32),
            pltpu.VMEM((MAXT, CHUNK), jnp.int32),
            pltpu.VMEM((NBUF, CHUNK, d), jnp.float32),
            pltpu.SemaphoreType.DMA((NBUF,)),
        ],
    )


_agg_hid = _make_agg(D_HID // NC)
_agg_out = _make_agg(D_OUT // NC)


_BN = 1024


def _tc1_body(x_ref, w1i_ref, w1r_ref, b1_ref, degp_ref,
              h0s_ref, root1_ref, dinv_ref):
    x = x_ref[...]
    deg = jnp.sum(degp_ref[...], axis=0)
    dinv = jnp.where(deg > 0, lax.rsqrt(jnp.maximum(deg, 1.0)), 0.0)
    h0 = jnp.dot(x, w1i_ref[...], preferred_element_type=jnp.float32)
    h0s = h0 * dinv[:, None]
    h0s_ref[0] = h0s[:, : D_HID // NC]
    h0s_ref[1] = h0s[:, D_HID // NC :]
    root1_ref[...] = (
        jnp.dot(x, w1r_ref[...], preferred_element_type=jnp.float32)
        + b1_ref[...][None, :]
    )
    dinv_ref[...] = dinv


def _tc1_call(x, w1i, w1r, b1, deg_parts):
    return pl.pallas_call(
        _tc1_body,
        grid=(pl.cdiv(N, _BN),),
        in_specs=[
            pl.BlockSpec((_BN, D_IN), lambda i: (i, 0)),
            pl.BlockSpec((D_IN, D_HID), lambda i: (0, 0)),
            pl.BlockSpec((D_IN, D_HID), lambda i: (0, 0)),
            pl.BlockSpec((D_HID,), lambda i: (0,)),
            pl.BlockSpec((NW, _BN), lambda i: (0, i)),
        ],
        out_specs=[
            pl.BlockSpec((NC, _BN, D_HID // NC), lambda i: (0, i, 0)),
            pl.BlockSpec((_BN, D_HID), lambda i: (i, 0)),
            pl.BlockSpec((_BN,), lambda i: (i,)),
        ],
        out_shape=[
            jax.ShapeDtypeStruct((NC, N, D_HID // NC), jnp.float32),
            jax.ShapeDtypeStruct((N, D_HID), jnp.float32),
            jax.ShapeDtypeStruct((N,), jnp.float32),
        ],
    )(x, w1i, w1r, b1, deg_parts)


def _tc2_body(agg_ref, root1_ref, dinv_ref, w2i_ref, w2r_ref, b2_ref,
              h1s_ref, root2_ref):
    agg = jnp.concatenate([agg_ref[0], agg_ref[1]], axis=-1)
    dinv = dinv_ref[...]
    out1 = jnp.maximum(agg * dinv[:, None] + root1_ref[...], 0.0)
    h1 = jnp.dot(out1, w2i_ref[...], preferred_element_type=jnp.float32)
    h1s = h1 * dinv[:, None]
    h1s_ref[0] = h1s[:, : D_OUT // NC]
    h1s_ref[1] = h1s[:, D_OUT // NC :]
    root2_ref[...] = (
        jnp.dot(out1, w2r_ref[...], preferred_element_type=jnp.float32)
        + b2_ref[...][None, :]
    )


def _tc2_call(agg, root1, dinv, w2i, w2r, b2):
    return pl.pallas_call(
        _tc2_body,
        grid=(pl.cdiv(N, _BN),),
        in_specs=[
            pl.BlockSpec((NC, _BN, D_HID // NC), lambda i: (0, i, 0)),
            pl.BlockSpec((_BN, D_HID), lambda i: (i, 0)),
            pl.BlockSpec((_BN,), lambda i: (i,)),
            pl.BlockSpec((D_HID, D_OUT), lambda i: (0, 0)),
            pl.BlockSpec((D_HID, D_OUT), lambda i: (0, 0)),
            pl.BlockSpec((D_OUT,), lambda i: (0,)),
        ],
        out_specs=[
            pl.BlockSpec((NC, _BN, D_OUT // NC), lambda i: (0, i, 0)),
            pl.BlockSpec((_BN, D_OUT), lambda i: (i, 0)),
        ],
        out_shape=[
            jax.ShapeDtypeStruct((NC, N, D_OUT // NC), jnp.float32),
            jax.ShapeDtypeStruct((N, D_OUT), jnp.float32),
        ],
    )(agg, root1, dinv, w2i, w2r, b2)


def _tc3_body(agg_ref, root2_ref, dinv_ref, out_ref):
    agg = jnp.concatenate([agg_ref[0], agg_ref[1]], axis=-1)
    out_ref[...] = jnp.maximum(
        agg * dinv_ref[...][:, None] + root2_ref[...], 0.0)


def _tc3_call(agg, root2, dinv):
    return pl.pallas_call(
        _tc3_body,
        grid=(pl.cdiv(N, _BN),),
        in_specs=[
            pl.BlockSpec((NC, _BN, D_OUT // NC), lambda i: (0, i, 0)),
            pl.BlockSpec((_BN, D_OUT), lambda i: (i, 0)),
            pl.BlockSpec((_BN,), lambda i: (i,)),
        ],
        out_specs=pl.BlockSpec((_BN, D_OUT), lambda i: (i, 0)),
        out_shape=jax.ShapeDtypeStruct((N, D_OUT), jnp.float32),
    )(agg, root2, dinv)


def kernel(x, edge_index, W1_init, W1_root, b1, W2_init, W2_root, b2):
    row = edge_index[0]
    col = edge_index[1]
    rowc = row.reshape(NCH, CHUNK)
    colc = col.reshape(NCH, CHUNK)
    zeros_hid = jnp.zeros((N, D_HID // NC), jnp.float32)
    zeros_out = jnp.zeros((N, D_OUT // NC), jnp.float32)

    deg_parts = _deg_call(col)
    h0s, root1, dinv = _tc1_call(x, W1_init, W1_root, b1, deg_parts)
    agg1 = _agg_hid(h0s, rowc, colc, zeros_hid)
    h1s, root2 = _tc2_call(agg1, root1, dinv, W2_init, W2_root, b2)
    agg2 = _agg_out(h1s, rowc, colc, zeros_out)
    return _tc3_call(agg2, root2, dinv)
